# Initial kernel scaffold; baseline (speedup 1.0000x reference)
import jax
import jax.numpy as jnp
from jax.experimental import pallas as pl


def kernel(h, e, eigvecs, eigvals, W_h, b_h, W_pe, b_pe, Wq_pe, Wk_pe, Wv_pe, Wo_pe, Wq, Wk, Wv, Wo, bn1_g, bn1_b, W1, b1, W2, b2, bn2_g, bn2_b, Wr1, br1, Wr2, br2, Wr3, br3, edge_index):
    raise NotImplementedError("write your pallas kernel here")


# speedup vs baseline: 1.0528x; 1.0528x over previous
import functools

import jax
import jax.numpy as jnp
import numpy as np
from jax.experimental import pallas as pl
from jax.experimental.pallas import tpu as pltpu

N = 10000
E = 320000
IN = 128
HID = 128
LPE = 16
LPE_H = 4
NL = 4
NH = 8
DH = 16
FF = 256
NC = 10


def _readout_body(x_ref, Wr1_ref, br1_ref, Wr2_ref, br2_ref, Wr3_ref, br3_ref,
                  out_ref, acc_ref):
    i = pl.program_id(0)
    n = pl.num_programs(0)

    @pl.when(i == 0)
    def _():
        acc_ref[...] = jnp.zeros_like(acc_ref)

    acc_ref[...] += jnp.sum(x_ref[...], axis=0, keepdims=True)

    @pl.when(i == n - 1)
    def _():
        hg = acc_ref[...] / N
        z = jax.nn.relu(hg @ Wr1_ref[...] + br1_ref[...][None, :])
        z = jax.nn.relu(z @ Wr2_ref[...] + br2_ref[...][None, :])
        out_ref[...] = z @ Wr3_ref[...] + br3_ref[...][None, :]


def _readout(x, Wr1, br1, Wr2, br2, Wr3, br3):
    BLK = 2000
    grid = (N // BLK,)
    return pl.pallas_call(
        _readout_body,
        grid=grid,
        in_specs=[
            pl.BlockSpec((BLK, HID), lambda i: (i, 0)),
            pl.BlockSpec((HID, 64), lambda i: (0, 0)),
            pl.BlockSpec((64,), lambda i: (0,)),
            pl.BlockSpec((64, 32), lambda i: (0, 0)),
            pl.BlockSpec((32,), lambda i: (0,)),
            pl.BlockSpec((32, NC), lambda i: (0, 0)),
            pl.BlockSpec((NC,), lambda i: (0,)),
        ],
        out_specs=pl.BlockSpec((1, NC), lambda i: (0, 0)),
        out_shape=jax.ShapeDtypeStruct((1, NC), jnp.float32),
        scratch_shapes=[pltpu.VMEM((1, HID), jnp.float32)],
    )(x, Wr1, br1, Wr2, br2, Wr3, br3)


def kernel(h, e, eigvecs, eigvals, W_h, b_h, W_pe, b_pe, Wq_pe, Wk_pe, Wv_pe,
           Wo_pe, Wq, Wk, Wv, Wo, bn1_g, bn1_b, W1, b1, W2, b2, bn2_g, bn2_b,
           Wr1, br1, Wr2, br2, Wr3, br3, edge_index):
    h0 = h @ W_h + b_h
    pe = jnp.stack([eigvecs, eigvals], axis=-1) @ W_pe + b_pe
    dh = LPE // LPE_H
    q = (pe @ Wq_pe).reshape(N, LPE, LPE_H, dh)
    k = (pe @ Wk_pe).reshape(N, LPE, LPE_H, dh)
    v = (pe @ Wv_pe).reshape(N, LPE, LPE_H, dh)
    att = jax.nn.softmax(jnp.einsum('nqhd,nkhd->nhqk', q, k) / np.sqrt(dh), axis=-1)
    o = jnp.einsum('nhqk,nkhd->nqhd', att, v).reshape(N, LPE, LPE)
    pe = pe + o @ Wo_pe
    x = jnp.concatenate([h0, pe.sum(axis=1)], axis=-1)
    src, dst = edge_index[0], edge_index[1]
    for i in range(NL):
        qn = (x @ Wq[i]).reshape(N, NH, DH)
        kn = (x @ Wk[i]).reshape(N, NH, DH)
        vn = (x @ Wv[i]).reshape(N, NH, DH)
        sc = jnp.clip(jnp.sum(qn[dst] * kn[src], axis=-1) / np.sqrt(DH), -5.0, 5.0)
        ex = jnp.exp(sc)
        den = jax.ops.segment_sum(ex, dst, num_segments=N) + 1e-6
        num = jax.ops.segment_sum(ex[:, :, None] * vn[src], dst, num_segments=N)
        agg = num / den[:, :, None]
        x1 = x + agg.reshape(N, HID) @ Wo[i]
        mu = x1.mean(0); var = x1.var(0)
        x1 = (x1 - mu) / jnp.sqrt(var + 1e-5) * bn1_g[i] + bn1_b[i]
        f = jax.nn.relu(x1 @ W1[i] + b1[i]) @ W2[i] + b2[i]
        x2 = x1 + f
        mu = x2.mean(0); var = x2.var(0)
        x = (x2 - mu) / jnp.sqrt(var + 1e-5) * bn2_g[i] + bn2_b[i]
    hg = x.mean(axis=0, keepdims=True)
    z = jax.nn.relu(hg @ Wr1 + br1)
    z = jax.nn.relu(z @ Wr2 + br2)
    return z @ Wr3 + br3


# speedup vs baseline: 6.0155x; 5.7137x over previous
import functools

import jax
import jax.numpy as jnp
import numpy as np
from jax import lax
from jax.experimental import pallas as pl
from jax.experimental.pallas import tpu as pltpu
from jax.experimental.pallas import tpu_sc as plsc

N = 10000
E = 320000
IN = 128
HID = 128
LPE = 16
LPE_H = 4
NL = 4
NH = 8
DH = 16
FF = 256
NC = 10

NW = 32
NPN = 320
NP = NW * NPN
EPW = E // NW
CG = 400
CGE = 2000
CF = 512
CW = 256
RPP = NP + 8

_mesh = plsc.VectorSubcoreMesh(core_axis_name="c", subcore_axis_name="s")


def _wid():
    return lax.axis_index("s") * 2 + lax.axis_index("c")


@functools.partial(
    pl.kernel, mesh=_mesh,
    out_type=[jax.ShapeDtypeStruct((E, HID), jnp.float32)] * 3,
    scratch_types=[
        pltpu.VMEM((CG,), jnp.int32),
        pltpu.VMEM((CG,), jnp.int32),
        pltpu.VMEM((CG, HID), jnp.float32),
        pltpu.SemaphoreType.DMA,
    ],
)
def _gather_qkv(q_hbm, k_hbm, v_hbm, dst_hbm, src_hbm,
                qd_hbm, ks_hbm, vs_hbm, idx_d, idx_s, rows, sem):
    base0 = _wid() * EPW

    def chunk(i, _):
        base = base0 + i * CG
        pltpu.sync_copy(dst_hbm.at[pl.ds(base, CG)], idx_d)
        pltpu.sync_copy(src_hbm.at[pl.ds(base, CG)], idx_s)
        pltpu.async_copy(q_hbm.at[idx_d], rows, sem).wait()
        pltpu.sync_copy(rows, qd_hbm.at[pl.ds(base, CG)])
        pltpu.async_copy(k_hbm.at[idx_s], rows, sem).wait()
        pltpu.sync_copy(rows, ks_hbm.at[pl.ds(base, CG)])
        pltpu.async_copy(v_hbm.at[idx_s], rows, sem).wait()
        pltpu.sync_copy(rows, vs_hbm.at[pl.ds(base, CG)])
        return 0

    lax.fori_loop(0, EPW // CG, chunk, 0)


@functools.partial(
    pl.kernel, mesh=_mesh,
    out_type=jax.ShapeDtypeStruct((E, HID), jnp.float32),
    scratch_types=[
        pltpu.VMEM((CG,), jnp.int32),
        pltpu.VMEM((CG, HID), jnp.float32),
        pltpu.SemaphoreType.DMA,
    ],
)
def _gather_e128(tab_hbm, dst_hbm, out_hbm, idx_d, rows, sem):
    base0 = _wid() * EPW

    def chunk(i, _):
        base = base0 + i * CG
        pltpu.sync_copy(dst_hbm.at[pl.ds(base, CG)], idx_d)
        pltpu.async_copy(tab_hbm.at[idx_d], rows, sem).wait()
        pltpu.sync_copy(rows, out_hbm.at[pl.ds(base, CG)])
        return 0

    lax.fori_loop(0, EPW // CG, chunk, 0)


def _load_bounds(rp_hbm, rp_v, lo):
    pltpu.sync_copy(rp_hbm.at[pl.ds(lo, NPN + 8)], rp_v)
    start = rp_v[pl.ds(0, 16)][0]
    end = rp_v[pl.ds(NPN - 8, 16)][8]
    return start, end


def _fold16(op, eps, rp_hbm, dst_hbm, x_hbm, out_hbm, rp_v, dstc, xc, outl):
    wid = _wid()
    lo = wid * NPN
    start, end = _load_bounds(rp_hbm, rp_v, lo)

    def init(r, _):
        outl[pl.ds(r * 16, 16)] = jnp.zeros(16, jnp.float32)
        return 0
    lax.fori_loop(0, NPN, init, 0)

    astart = (start // 8) * 8
    nchunk = lax.div(end - astart + CF - 1, CF)

    def chunk(i, carry):
        cbase = astart + i * CF
        pltpu.sync_copy(dst_hbm.at[pl.ds(cbase, CF)], dstc)
        pltpu.sync_copy(x_hbm.at[pl.ds(cbase * 16, CF * 16)], xc)
        jlo = jnp.maximum(0, start - cbase)
        jhi = jnp.minimum(CF, end - cbase)

        def group(g, carry):
            cur, acc = carry
            ids = dstc[pl.ds(g * 16, 16)]
            for l in range(16):
                j = g * 16 + l
                nd = ids[l]
                active = (j >= jlo) & (j < jhi)
                row = xc[pl.ds(j * 16, 16)]
                is_new = active & (nd != cur)

                @pl.when(is_new & (cur >= 0))
                def _(acc=acc, cur=cur):
                    outl[pl.ds((cur - lo) * 16, 16)] = acc

                acc = jnp.where(is_new, row,
                                jnp.where(active, op(acc, row), acc))
                cur = jnp.where(active, nd, cur)
            return cur, acc

        return lax.fori_loop(jlo // 16, lax.div(jhi + 15, 16), group, carry)

    cur, acc = lax.fori_loop(0, nchunk, chunk,
                             (-1, jnp.zeros(16, jnp.float32)))

    @pl.when(cur >= 0)
    def _():
        outl[pl.ds((cur - lo) * 16, 16)] = acc

    if eps:
        def addeps(r, _):
            outl[pl.ds(r * 16, 16)] = outl[pl.ds(r * 16, 16)] + 1e-6
            return 0
        lax.fori_loop(0, NPN, addeps, 0)
    pltpu.sync_copy(outl, out_hbm.at[pl.ds(lo * 16, NPN * 16)])


_fold16_types = dict(
    out_type=jax.ShapeDtypeStruct((NP * 16,), jnp.float32),
    scratch_types=[
        pltpu.VMEM((NPN + 8,), jnp.int32),
        pltpu.VMEM((CF,), jnp.int32),
        pltpu.VMEM((CF * 16,), jnp.float32),
        pltpu.VMEM((NPN * 16,), jnp.float32),
    ],
)


@functools.partial(pl.kernel, mesh=_mesh, **_fold16_types)
def _segmax16(rp_hbm, dst_hbm, x_hbm, out_hbm, rp_v, dstc, xc, outl):
    _fold16(jnp.maximum, False, rp_hbm, dst_hbm, x_hbm, out_hbm,
            rp_v, dstc, xc, outl)


@functools.partial(pl.kernel, mesh=_mesh, **_fold16_types)
def _segsum16(rp_hbm, dst_hbm, x_hbm, out_hbm, rp_v, dstc, xc, outl):
    _fold16(lax.add, True, rp_hbm, dst_hbm, x_hbm, out_hbm,
            rp_v, dstc, xc, outl)


@functools.partial(
    pl.kernel, mesh=_mesh,
    out_type=jax.ShapeDtypeStruct((NP * HID,), jnp.float32),
    scratch_types=[
        pltpu.VMEM((NPN + 8,), jnp.int32),
        pltpu.VMEM((CW,), jnp.int32),
        pltpu.VMEM((CW * HID,), jnp.float32),
        pltpu.VMEM((NPN * HID,), jnp.float32),
    ],
)
def _segsum128(rp_hbm, dst_hbm, x_hbm, out_hbm, rp_v, dstc, xc, outl):
    wid = _wid()
    lo = wid * NPN
    start, end = _load_bounds(rp_hbm, rp_v, lo)

    def init(r, _):
        for t in range(8):
            outl[pl.ds(r * HID + t * 16, 16)] = jnp.zeros(16, jnp.float32)
        return 0
    lax.fori_loop(0, NPN, init, 0)

    astart = (start // 8) * 8
    nchunk = lax.div(end - astart + CW - 1, CW)
    zero = jnp.zeros(16, jnp.float32)

    def chunk(i, carry):
        cbase = astart + i * CW
        pltpu.sync_copy(dst_hbm.at[pl.ds(cbase, CW)], dstc)
        pltpu.sync_copy(x_hbm.at[pl.ds(cbase * HID, CW * HID)], xc)
        jlo = jnp.maximum(0, start - cbase)
        jhi = jnp.minimum(CW, end - cbase)

        def group(g, carry):
            cur = carry[0]
            acc = list(carry[1:])
            ids = dstc[pl.ds(g * 16, 16)]
            for l in range(16):
                j = g * 16 + l
                nd = ids[l]
                active = (j >= jlo) & (j < jhi)
                is_new = active & (nd != cur)

                @pl.when(is_new & (cur >= 0))
                def _(acc=tuple(acc), cur=cur):
                    for t in range(8):
                        outl[pl.ds((cur - lo) * HID + t * 16, 16)] = acc[t]

                for t in range(8):
                    row_t = xc[pl.ds(j * HID + t * 16, 16)]
                    acc[t] = jnp.where(
                        is_new, row_t,
                        jnp.where(active, acc[t] + row_t, acc[t]))
                cur = jnp.where(active, nd, cur)
            return (cur,) + tuple(acc)

        return lax.fori_loop(jlo // 16, lax.div(jhi + 15, 16), group, carry)

    carry = lax.fori_loop(0, nchunk, chunk, (-1,) + (zero,) * 8)
    cur = carry[0]

    @pl.when(cur >= 0)
    def _():
        for t in range(8):
            outl[pl.ds((cur - lo) * HID + t * 16, 16)] = carry[1 + t]

    pltpu.sync_copy(outl, out_hbm.at[pl.ds(lo * HID, NPN * HID)])


def kernel(h, e, eigvecs, eigvals, W_h, b_h, W_pe, b_pe, Wq_pe, Wk_pe, Wv_pe,
           Wo_pe, Wq, Wk, Wv, Wo, bn1_g, bn1_b, W1, b1, W2, b2, bn2_g, bn2_b,
           Wr1, br1, Wr2, br2, Wr3, br3, edge_index):
    h0 = h @ W_h + b_h
    pe = jnp.stack([eigvecs, eigvals], axis=-1) @ W_pe + b_pe
    dh = LPE // LPE_H
    q = (pe @ Wq_pe).reshape(N, LPE, LPE_H, dh)
    k = (pe @ Wk_pe).reshape(N, LPE, LPE_H, dh)
    v = (pe @ Wv_pe).reshape(N, LPE, LPE_H, dh)
    att = jax.nn.softmax(jnp.einsum('nqhd,nkhd->nhqk', q, k) / np.sqrt(dh), axis=-1)
    o = jnp.einsum('nhqk,nkhd->nqhd', att, v).reshape(N, LPE, LPE)
    pe = pe + o @ Wo_pe
    x = jnp.concatenate([h0, pe.sum(axis=1)], axis=-1)
    src, dst = edge_index[0], edge_index[1]

    order = jnp.argsort(dst, stable=True)
    dst_s = dst[order]
    src_s = src[order]
    row_ptr = jnp.searchsorted(dst_s, jnp.arange(NP + 1), side='left')
    row_ptr = jnp.concatenate(
        [row_ptr, jnp.full((RPP - NP - 1,), E)]).astype(jnp.int32)
    dst_pad = jnp.concatenate(
        [dst_s, jnp.full((CF,), NP, dst_s.dtype)]).astype(jnp.int32)

    for i in range(NL):
        qn = x @ Wq[i]
        kn = x @ Wk[i]
        vn = x @ Wv[i]
        qd, ks, vs = _gather_qkv(qn, kn, vn, dst_s, src_s)
        qd3 = qd.reshape(E, NH, DH)
        ks3 = ks.reshape(E, NH, DH)
        dot = qd3 * ks3
        _w = DH
        while _w > 1:
            _w //= 2
            dot = dot[..., 0::2] + dot[..., 1::2]
        dot = dot[..., 0]
        sc = jnp.clip(dot / np.sqrt(DH), -5.0, 5.0)
        sc16 = jnp.pad(sc, ((0, CF), (0, 8))).reshape(-1)
        smax = _segmax16(row_ptr, dst_pad, sc16).reshape(NP, 16)
        smax_e = _gather_e128(jnp.pad(smax, ((0, 0), (0, HID - 16))),
                              dst_s)[:, :8]
        ex = jnp.exp(sc - smax_e)
        ex16 = jnp.pad(ex, ((0, CF), (0, 8))).reshape(-1)
        den = _segsum16(row_ptr, dst_pad, ex16).reshape(NP, 16)
        den_e = _gather_e128(jnp.pad(den, ((0, 0), (0, HID - 16))),
                             dst_s)[:, :8]
        wv = ((ex / den_e)[:, :, None] * vs.reshape(E, NH, DH)).reshape(E, HID)
        wv = jnp.pad(wv, ((0, CW), (0, 0))).reshape(-1)
        agg = _segsum128(row_ptr, dst_pad, wv).reshape(NP, HID)[:N]
        x1 = x + agg @ Wo[i]
        mu = x1.mean(0); var = x1.var(0)
        x1 = (x1 - mu) / jnp.sqrt(var + 1e-5) * bn1_g[i] + bn1_b[i]
        f = jax.nn.relu(x1 @ W1[i] + b1[i]) @ W2[i] + b2[i]
        x2 = x1 + f
        mu = x2.mean(0); var = x2.var(0)
        x = (x2 - mu) / jnp.sqrt(var + 1e-5) * bn2_g[i] + bn2_b[i]
    hg = x.mean(axis=0, keepdims=True)
    z = jax.nn.relu(hg @ Wr1 + br1)
    z = jax.nn.relu(z @ Wr2 + br2)
    return z @ Wr3 + br3
